# baseline (speedup 1.0000x reference)
import jax
import jax.numpy as jnp
from jax import lax
from jax.experimental import pallas as pl
from jax.experimental.pallas import tpu as pltpu
from jax.experimental.pallas import tpu_sc as plsc

_N_CLS = 1000
_FEAT = 512
_BATCH = 1024
_M = 0.95
_TEMP = 0.1
_BASE_TEMP = 0.1

_NW = 32
_NS = 16
_PAD_CLS = 1024
_RPW = _PAD_CLS // _NW
_LAST_RPW = _N_CLS - (_NW - 1) * _RPW
_L = 16
_SROWS = _BATCH // _NS


def _sc_ema_body(feat_hbm, protos_hbm, labels_hbm, out_hbm,
                 labels_v, midx, mrow, xrow, ptile,
                 semA, semB, semP):
    cid = lax.axis_index("c")
    sid = lax.axis_index("s")
    wid = sid * 2 + cid
    base = wid * _RPW

    @pl.when(wid < _NW - 1)
    def _():
        pltpu.make_async_copy(protos_hbm.at[pl.ds(base, _RPW)], ptile, semP).start()

    @pl.when(wid == _NW - 1)
    def _():
        pltpu.make_async_copy(protos_hbm.at[pl.ds(base, _LAST_RPW)],
                              ptile.at[pl.ds(0, _LAST_RPW)], semP).start()

    pltpu.sync_copy(labels_hbm, labels_v.at[pl.ds(0, _BATCH)])

    iota16 = lax.iota(jnp.int32, _L)

    def scan_g(g, off_c):
        lab16 = labels_v[pl.ds(g * _L, _L)]
        rel = lab16 - base
        m = jnp.logical_and(rel >= 0, rel < _RPW)
        plsc.store_compressed(midx.at[pl.ds(off_c, _L)], iota16 + g * _L, mask=m)
        plsc.store_compressed(mrow.at[pl.ds(off_c, _L)], rel, mask=m)
        return off_c + plsc.all_reduce_population_count(m)[0]

    off = lax.fori_loop(0, _BATCH // _L, scan_g, jnp.int32(0), unroll=False)

    @pl.when(wid < _NW - 1)
    def _():
        pltpu.make_async_copy(protos_hbm.at[pl.ds(base, _RPW)], ptile, semP).wait()

    @pl.when(wid == _NW - 1)
    def _():
        pltpu.make_async_copy(protos_hbm.at[pl.ds(base, _LAST_RPW)],
                              ptile.at[pl.ds(0, _LAST_RPW)], semP).wait()

    def fetch(s, buf, sem_b):
        i = midx[pl.ds(s, _L)][0]
        pltpu.make_async_copy(feat_hbm.at[i], xrow.at[buf], sem_b).start()

    @pl.when(off > 0)
    def _():
        fetch(0, 0, semA)

    @pl.when(off > 1)
    def _():
        fetch(1, 1, semB)

    def upd(s, buf, sem_b):
        pltpu.make_async_copy(feat_hbm.at[0], xrow.at[buf], sem_b).wait()
        r = mrow[pl.ds(s, _L)][0]
        accs = [jnp.zeros((_L,), jnp.float32) for _ in range(4)]
        ys = []
        for j in range(_FEAT // _L):
            pj = ptile[r, pl.ds(j * _L, _L)]
            xj = xrow[buf, pl.ds(j * _L, _L)]
            y = pj * _M + xj * (1.0 - _M)
            ys.append(y)
            accs[j % 4] = accs[j % 4] + y * y
        ss = jnp.sum((accs[0] + accs[1]) + (accs[2] + accs[3]))
        sv = jnp.full((_L,), ss, jnp.float32)
        iv = plsc.bitcast(sv, jnp.int32)
        y0 = plsc.bitcast(jnp.int32(0x5F3759DF) - (iv >> 1), jnp.float32)
        y = y0
        for _ in range(4):
            y = y * (1.5 - 0.5 * sv * y * y)
        normv = sv * y
        scale = jnp.where(normv > 1e-12, y, 1e12)
        for j in range(_FEAT // _L):
            ptile[r, pl.ds(j * _L, _L)] = ys[j] * scale

        @pl.when(s + 2 < off)
        def _():
            fetch(s + 2, buf, sem_b)

    def pair(t, carry):
        s0 = 2 * t
        upd(s0, 0, semA)

        @pl.when(s0 + 1 < off)
        def _():
            upd(s0 + 1, 1, semB)

        return carry

    lax.fori_loop(0, (off + 1) // 2, pair, 0, unroll=False)

    @pl.when(wid < _NW - 1)
    def _():
        pltpu.sync_copy(ptile, out_hbm.at[pl.ds(base, _RPW)])

    @pl.when(wid == _NW - 1)
    def _():
        pltpu.sync_copy(ptile.at[pl.ds(0, _LAST_RPW)],
                        out_hbm.at[pl.ds(base, _LAST_RPW)])


def _loss_body(protos_ref, out_ref):
    protos = protos_ref[...]
    hi = protos.astype(jnp.bfloat16)
    lo = (protos - hi.astype(jnp.float32)).astype(jnp.bfloat16)
    dims = (((1,), (1,)), ((), ()))
    dot = lambda a, b: lax.dot_general(a, b, dims,
                                       preferred_element_type=jnp.float32)
    logits = (dot(hi, hi) + (dot(hi, lo) + dot(lo, hi))) * (1.0 / _TEMP)
    e = jnp.exp(logits)
    r = lax.broadcasted_iota(jnp.int32, (_N_CLS, _N_CLS), 0)
    c = lax.broadcasted_iota(jnp.int32, (_N_CLS, _N_CLS), 1)
    e = jnp.where(r == c, 0.0, e)
    s = jnp.sum(e, axis=1)
    mpn = jnp.log(s * (1.0 / (_N_CLS - 1)))
    out_ref[0, 0] = (_TEMP / _BASE_TEMP) * jnp.sum(mpn) * (1.0 / _N_CLS)


def kernel(features, prototypes, labels):
    ema = pl.kernel(
        _sc_ema_body,
        out_type=jax.ShapeDtypeStruct((_N_CLS, _FEAT), jnp.float32),
        mesh=plsc.VectorSubcoreMesh(core_axis_name="c", subcore_axis_name="s"),
        compiler_params=pltpu.CompilerParams(needs_layout_passes=False),
        scratch_types=[
            pltpu.VMEM((_BATCH + _L,), jnp.int32),
            pltpu.VMEM((_BATCH + _L,), jnp.int32),
            pltpu.VMEM((_BATCH + _L,), jnp.int32),
            pltpu.VMEM((2, _FEAT), jnp.float32),
            pltpu.VMEM((_RPW, _FEAT), jnp.float32),
            pltpu.SemaphoreType.DMA,
            pltpu.SemaphoreType.DMA,
            pltpu.SemaphoreType.DMA,
        ],
    )
    protos_new = ema(features, prototypes, labels)

    out = pl.pallas_call(
        _loss_body,
        out_shape=jax.ShapeDtypeStruct((1, 1), jnp.float32),
        in_specs=[pl.BlockSpec(memory_space=pltpu.VMEM)],
        out_specs=pl.BlockSpec(memory_space=pltpu.SMEM),
    )(protos_new)
    return out[0, 0]


# speedup vs baseline: 1.0997x; 1.0997x over previous
import jax
import jax.numpy as jnp
from jax import lax
from jax.experimental import pallas as pl
from jax.experimental.pallas import tpu as pltpu
from jax.experimental.pallas import tpu_sc as plsc

_N_CLS = 1000
_FEAT = 512
_BATCH = 1024
_M = 0.95
_TEMP = 0.1
_BASE_TEMP = 0.1

_NW = 32
_NS = 16
_PAD_CLS = 1024
_RPW = _PAD_CLS // _NW
_LAST_RPW = _N_CLS - (_NW - 1) * _RPW
_L = 16
_SROWS = _BATCH // _NS


def _sc_ema_body(feat_hbm, protos_hbm, labels_hbm, out_hbm,
                 labels_v, midx, mrow, xrow, ptile, feats_sh, sem,
                 semA, semB, semP):
    cid = lax.axis_index("c")
    sid = lax.axis_index("s")
    wid = sid * 2 + cid
    base = wid * _RPW

    stage = pltpu.async_copy(
        feat_hbm.at[pl.ds(sid * _SROWS, _SROWS)],
        feats_sh.at[pl.ds(sid * _SROWS, _SROWS)], sem)

    @pl.when(wid < _NW - 1)
    def _():
        pltpu.make_async_copy(protos_hbm.at[pl.ds(base, _RPW)], ptile, semP).start()

    @pl.when(wid == _NW - 1)
    def _():
        pltpu.make_async_copy(protos_hbm.at[pl.ds(base, _LAST_RPW)],
                              ptile.at[pl.ds(0, _LAST_RPW)], semP).start()

    pltpu.sync_copy(labels_hbm, labels_v.at[pl.ds(0, _BATCH)])

    iota16 = lax.iota(jnp.int32, _L)

    def scan_g(g, off_c):
        lab16 = labels_v[pl.ds(g * _L, _L)]
        rel = lab16 - base
        m = jnp.logical_and(rel >= 0, rel < _RPW)
        plsc.store_compressed(midx.at[pl.ds(off_c, _L)], iota16 + g * _L, mask=m)
        plsc.store_compressed(mrow.at[pl.ds(off_c, _L)], rel, mask=m)
        return off_c + plsc.all_reduce_population_count(m)[0]

    off = lax.fori_loop(0, _BATCH // _L, scan_g, jnp.int32(0), unroll=False)

    stage.wait()
    plsc.subcore_barrier()

    @pl.when(wid < _NW - 1)
    def _():
        pltpu.make_async_copy(protos_hbm.at[pl.ds(base, _RPW)], ptile, semP).wait()

    @pl.when(wid == _NW - 1)
    def _():
        pltpu.make_async_copy(protos_hbm.at[pl.ds(base, _LAST_RPW)],
                              ptile.at[pl.ds(0, _LAST_RPW)], semP).wait()

    def fetch(s, buf, sem_b):
        i = midx[pl.ds(s, _L)][0]
        pltpu.make_async_copy(feats_sh.at[i], xrow.at[buf], sem_b).start()

    @pl.when(off > 0)
    def _():
        fetch(0, 0, semA)

    @pl.when(off > 1)
    def _():
        fetch(1, 1, semB)

    def upd(s, buf, sem_b):
        pltpu.make_async_copy(feat_hbm.at[0], xrow.at[buf], sem_b).wait()
        r = mrow[pl.ds(s, _L)][0]
        accs = [jnp.zeros((_L,), jnp.float32) for _ in range(4)]
        ys = []
        for j in range(_FEAT // _L):
            pj = ptile[r, pl.ds(j * _L, _L)]
            xj = xrow[buf, pl.ds(j * _L, _L)]
            y = pj * _M + xj * (1.0 - _M)
            ys.append(y)
            accs[j % 4] = accs[j % 4] + y * y
        ss = jnp.sum((accs[0] + accs[1]) + (accs[2] + accs[3]))
        sv = jnp.full((_L,), ss, jnp.float32)
        iv = plsc.bitcast(sv, jnp.int32)
        y0 = plsc.bitcast(jnp.int32(0x5F3759DF) - (iv >> 1), jnp.float32)
        y = y0
        for _ in range(4):
            y = y * (1.5 - 0.5 * sv * y * y)
        normv = sv * y
        scale = jnp.where(normv > 1e-12, y, 1e12)
        for j in range(_FEAT // _L):
            ptile[r, pl.ds(j * _L, _L)] = ys[j] * scale

        @pl.when(s + 2 < off)
        def _():
            fetch(s + 2, buf, sem_b)

    def pair(t, carry):
        s0 = 2 * t
        upd(s0, 0, semA)

        @pl.when(s0 + 1 < off)
        def _():
            upd(s0 + 1, 1, semB)

        return carry

    lax.fori_loop(0, (off + 1) // 2, pair, 0, unroll=False)

    @pl.when(wid < _NW - 1)
    def _():
        pltpu.sync_copy(ptile, out_hbm.at[pl.ds(base, _RPW)])

    @pl.when(wid == _NW - 1)
    def _():
        pltpu.sync_copy(ptile.at[pl.ds(0, _LAST_RPW)],
                        out_hbm.at[pl.ds(base, _LAST_RPW)])


_RB = 200


def _loss_body(lhs_ref, rhs_ref, out_ref):
    i = pl.program_id(0)
    lhs = lhs_ref[...]
    rhs = rhs_ref[...]
    lhi = lhs.astype(jnp.bfloat16)
    llo = (lhs - lhi.astype(jnp.float32)).astype(jnp.bfloat16)
    rhi = rhs.astype(jnp.bfloat16)
    rlo = (rhs - rhi.astype(jnp.float32)).astype(jnp.bfloat16)
    dims = (((1,), (1,)), ((), ()))
    dot = lambda a, b: lax.dot_general(a, b, dims,
                                       preferred_element_type=jnp.float32)
    logits = (dot(lhi, rhi) + (dot(lhi, rlo) + dot(llo, rhi))) * (1.0 / _TEMP)
    e = jnp.exp(logits)
    r = lax.broadcasted_iota(jnp.int32, (_RB, _N_CLS), 0) + i * _RB
    c = lax.broadcasted_iota(jnp.int32, (_RB, _N_CLS), 1)
    e = jnp.where(r == c, 0.0, e)
    s = jnp.sum(e, axis=1)
    mpn = jnp.log(s * (1.0 / (_N_CLS - 1)))
    part = (_TEMP / _BASE_TEMP) * jnp.sum(mpn) * (1.0 / _N_CLS)

    @pl.when(i == 0)
    def _():
        out_ref[0, 0] = 0.0

    out_ref[0, 0] += part


def kernel(features, prototypes, labels):
    ema = pl.kernel(
        _sc_ema_body,
        out_type=jax.ShapeDtypeStruct((_N_CLS, _FEAT), jnp.float32),
        mesh=plsc.VectorSubcoreMesh(core_axis_name="c", subcore_axis_name="s"),
        compiler_params=pltpu.CompilerParams(needs_layout_passes=False),
        scratch_types=[
            pltpu.VMEM((_BATCH + _L,), jnp.int32),
            pltpu.VMEM((_BATCH + _L,), jnp.int32),
            pltpu.VMEM((_BATCH + _L,), jnp.int32),
            pltpu.VMEM((2, _FEAT), jnp.float32),
            pltpu.VMEM((_RPW, _FEAT), jnp.float32),
            pltpu.VMEM_SHARED((_BATCH, _FEAT), jnp.float32),
            pltpu.SemaphoreType.DMA,
            pltpu.SemaphoreType.DMA,
            pltpu.SemaphoreType.DMA,
            pltpu.SemaphoreType.DMA,
        ],
    )
    protos_new = ema(features, prototypes, labels)

    out = pl.pallas_call(
        _loss_body,
        grid=(_N_CLS // _RB,),
        out_shape=jax.ShapeDtypeStruct((1, 1), jnp.float32),
        in_specs=[
            pl.BlockSpec((_RB, _FEAT), lambda i: (i, 0)),
            pl.BlockSpec((_N_CLS, _FEAT), lambda i: (0, 0)),
        ],
        out_specs=pl.BlockSpec(memory_space=pltpu.SMEM),
    )(protos_new, protos_new)
    return out[0, 0]


# speedup vs baseline: 1.2077x; 1.0982x over previous
import jax
import jax.numpy as jnp
from jax import lax
from jax.experimental import pallas as pl
from jax.experimental.pallas import tpu as pltpu
from jax.experimental.pallas import tpu_sc as plsc

_N_CLS = 1000
_FEAT = 512
_BATCH = 1024
_M = 0.95
_TEMP = 0.1
_BASE_TEMP = 0.1

_NW = 32
_NS = 16
_PAD_CLS = 1024
_RPW = _PAD_CLS // _NW
_LAST_RPW = _N_CLS - (_NW - 1) * _RPW
_L = 16
_SROWS = _BATCH // _NS


def _sc_ema_body(feat_hbm, protos_hbm, labels_hbm, out_hbm,
                 labels_v, midx, mrow, xrow, ptile, feats_sh, sem,
                 semA, semB, semP):
    cid = lax.axis_index("c")
    sid = lax.axis_index("s")
    wid = sid * 2 + cid
    base = wid * _RPW

    stage = pltpu.async_copy(
        feat_hbm.at[pl.ds(sid * _SROWS, _SROWS)],
        feats_sh.at[pl.ds(sid * _SROWS, _SROWS)], sem)

    @pl.when(wid < _NW - 1)
    def _():
        pltpu.make_async_copy(protos_hbm.at[pl.ds(base, _RPW)], ptile, semP).start()

    @pl.when(wid == _NW - 1)
    def _():
        pltpu.make_async_copy(protos_hbm.at[pl.ds(base, _LAST_RPW)],
                              ptile.at[pl.ds(0, _LAST_RPW)], semP).start()

    pltpu.sync_copy(labels_hbm, labels_v.at[pl.ds(0, _BATCH)])

    iota16 = lax.iota(jnp.int32, _L)

    def scan_g(g, off_c):
        lab16 = labels_v[pl.ds(g * _L, _L)]
        rel = lab16 - base
        m = jnp.logical_and(rel >= 0, rel < _RPW)
        plsc.store_compressed(midx.at[pl.ds(off_c, _L)], iota16 + g * _L, mask=m)
        plsc.store_compressed(mrow.at[pl.ds(off_c, _L)], rel, mask=m)
        return off_c + plsc.all_reduce_population_count(m)[0]

    off = lax.fori_loop(0, _BATCH // _L, scan_g, jnp.int32(0), unroll=False)

    stage.wait()
    plsc.subcore_barrier()

    @pl.when(wid < _NW - 1)
    def _():
        pltpu.make_async_copy(protos_hbm.at[pl.ds(base, _RPW)], ptile, semP).wait()

    @pl.when(wid == _NW - 1)
    def _():
        pltpu.make_async_copy(protos_hbm.at[pl.ds(base, _LAST_RPW)],
                              ptile.at[pl.ds(0, _LAST_RPW)], semP).wait()

    def fetch(s, buf, sem_b):
        i = midx[pl.ds(s, _L)][0]
        pltpu.make_async_copy(feats_sh.at[i], xrow.at[buf], sem_b).start()

    @pl.when(off > 0)
    def _():
        fetch(0, 0, semA)

    @pl.when(off > 1)
    def _():
        fetch(1, 1, semB)

    def upd(s, buf, sem_b):
        pltpu.make_async_copy(feat_hbm.at[0], xrow.at[buf], sem_b).wait()
        r = mrow[pl.ds(s, _L)][0]
        accs = [jnp.zeros((_L,), jnp.float32) for _ in range(4)]
        ys = []
        for j in range(_FEAT // _L):
            pj = ptile[r, pl.ds(j * _L, _L)]
            xj = xrow[buf, pl.ds(j * _L, _L)]
            y = pj * _M + xj * (1.0 - _M)
            ys.append(y)
            accs[j % 4] = accs[j % 4] + y * y
        ss = jnp.sum((accs[0] + accs[1]) + (accs[2] + accs[3]))
        sv = jnp.full((_L,), ss, jnp.float32)
        iv = plsc.bitcast(sv, jnp.int32)
        y0 = plsc.bitcast(jnp.int32(0x5F3759DF) - (iv >> 1), jnp.float32)
        y = y0
        for _ in range(4):
            y = y * (1.5 - 0.5 * sv * y * y)
        normv = sv * y
        scale = jnp.where(normv > 1e-12, y, 1e12)
        for j in range(_FEAT // _L):
            ptile[r, pl.ds(j * _L, _L)] = ys[j] * scale

        @pl.when(s + 2 < off)
        def _():
            fetch(s + 2, buf, sem_b)

    def pair(t, carry):
        s0 = 2 * t
        upd(s0, 0, semA)

        @pl.when(s0 + 1 < off)
        def _():
            upd(s0 + 1, 1, semB)

        return carry

    lax.fori_loop(0, (off + 1) // 2, pair, 0, unroll=False)

    @pl.when(wid < _NW - 1)
    def _():
        pltpu.sync_copy(ptile, out_hbm.at[pl.ds(base, _RPW)])

    @pl.when(wid == _NW - 1)
    def _():
        pltpu.sync_copy(ptile.at[pl.ds(0, _LAST_RPW)],
                        out_hbm.at[pl.ds(base, _LAST_RPW)])


def _loss_body(protos_ref, out_ref):
    protos = protos_ref[...]
    hi = protos.astype(jnp.bfloat16)
    lo = (protos - hi.astype(jnp.float32)).astype(jnp.bfloat16)
    dims = (((1,), (1,)), ((), ()))
    dot = lambda a, b: lax.dot_general(a, b, dims,
                                       preferred_element_type=jnp.float32)
    logits = (dot(hi, hi) + (dot(hi, lo) + dot(lo, hi))) * (1.0 / _TEMP)
    e = jnp.exp(logits)
    r = lax.broadcasted_iota(jnp.int32, (_N_CLS, _N_CLS), 0)
    c = lax.broadcasted_iota(jnp.int32, (_N_CLS, _N_CLS), 1)
    e = jnp.where(r == c, 0.0, e)
    s = jnp.sum(e, axis=1)
    mpn = jnp.log(s * (1.0 / (_N_CLS - 1)))
    out_ref[0, 0] = (_TEMP / _BASE_TEMP) * jnp.sum(mpn) * (1.0 / _N_CLS)


def kernel(features, prototypes, labels):
    ema = pl.kernel(
        _sc_ema_body,
        out_type=jax.ShapeDtypeStruct((_N_CLS, _FEAT), jnp.float32),
        mesh=plsc.VectorSubcoreMesh(core_axis_name="c", subcore_axis_name="s"),
        compiler_params=pltpu.CompilerParams(needs_layout_passes=False),
        scratch_types=[
            pltpu.VMEM((_BATCH + _L,), jnp.int32),
            pltpu.VMEM((_BATCH + _L,), jnp.int32),
            pltpu.VMEM((_BATCH + _L,), jnp.int32),
            pltpu.VMEM((2, _FEAT), jnp.float32),
            pltpu.VMEM((_RPW, _FEAT), jnp.float32),
            pltpu.VMEM_SHARED((_BATCH, _FEAT), jnp.float32),
            pltpu.SemaphoreType.DMA,
            pltpu.SemaphoreType.DMA,
            pltpu.SemaphoreType.DMA,
            pltpu.SemaphoreType.DMA,
        ],
    )
    protos_new = ema(features, prototypes, labels)

    out = pl.pallas_call(
        _loss_body,
        out_shape=jax.ShapeDtypeStruct((1, 1), jnp.float32),
        in_specs=[pl.BlockSpec(memory_space=pltpu.VMEM)],
        out_specs=pl.BlockSpec(memory_space=pltpu.SMEM),
    )(protos_new)
    return out[0, 0]


# speedup vs baseline: 1.2209x; 1.0109x over previous
import jax
import jax.numpy as jnp
from jax import lax
from jax.experimental import pallas as pl
from jax.experimental.pallas import tpu as pltpu
from jax.experimental.pallas import tpu_sc as plsc

_N_CLS = 1000
_FEAT = 512
_BATCH = 1024
_M = 0.95
_TEMP = 0.1
_BASE_TEMP = 0.1

_NW = 32
_NS = 16
_PAD_CLS = 1024
_RPW = _PAD_CLS // _NW
_LAST_RPW = _N_CLS - (_NW - 1) * _RPW
_L = 16
_SROWS = _BATCH // _NS


def _sc_ema_body(feat_hbm, protos_hbm, labels_hbm, out_hbm,
                 labels_v, midx, mrow, xrow, ptile, feats_sh, sem,
                 semA, semB, semP):
    cid = lax.axis_index("c")
    sid = lax.axis_index("s")
    wid = sid * 2 + cid
    base = wid * _RPW

    stage = pltpu.async_copy(
        feat_hbm.at[pl.ds(sid * _SROWS, _SROWS)],
        feats_sh.at[pl.ds(sid * _SROWS, _SROWS)], sem)

    @pl.when(wid < _NW - 1)
    def _():
        pltpu.make_async_copy(protos_hbm.at[pl.ds(base, _RPW)], ptile, semP).start()

    @pl.when(wid == _NW - 1)
    def _():
        pltpu.make_async_copy(protos_hbm.at[pl.ds(base, _LAST_RPW)],
                              ptile.at[pl.ds(0, _LAST_RPW)], semP).start()

    pltpu.sync_copy(labels_hbm, labels_v.at[pl.ds(0, _BATCH)])

    iota16 = lax.iota(jnp.int32, _L)

    def scan_g(g, off_c):
        lab16 = labels_v[pl.ds(g * _L, _L)]
        rel = lab16 - base
        m = jnp.logical_and(rel >= 0, rel < _RPW)
        plsc.store_compressed(midx.at[pl.ds(off_c, _L)], iota16 + g * _L, mask=m)
        plsc.store_compressed(mrow.at[pl.ds(off_c, _L)], rel, mask=m)
        return off_c + plsc.all_reduce_population_count(m)[0]

    off = lax.fori_loop(0, _BATCH // _L, scan_g, jnp.int32(0), unroll=4)

    stage.wait()
    plsc.subcore_barrier()

    @pl.when(wid < _NW - 1)
    def _():
        pltpu.make_async_copy(protos_hbm.at[pl.ds(base, _RPW)], ptile, semP).wait()

    @pl.when(wid == _NW - 1)
    def _():
        pltpu.make_async_copy(protos_hbm.at[pl.ds(base, _LAST_RPW)],
                              ptile.at[pl.ds(0, _LAST_RPW)], semP).wait()

    def fetch(s, buf, sem_b):
        i = midx[pl.ds(s, _L)][0]
        pltpu.make_async_copy(feats_sh.at[i], xrow.at[buf], sem_b).start()

    @pl.when(off > 0)
    def _():
        fetch(0, 0, semA)

    @pl.when(off > 1)
    def _():
        fetch(1, 1, semB)

    def upd(s, buf, sem_b):
        pltpu.make_async_copy(feat_hbm.at[0], xrow.at[buf], sem_b).wait()
        r = mrow[pl.ds(s, _L)][0]
        accs = [jnp.zeros((_L,), jnp.float32) for _ in range(4)]
        ys = []
        for j in range(_FEAT // _L):
            pj = ptile[r, pl.ds(j * _L, _L)]
            xj = xrow[buf, pl.ds(j * _L, _L)]
            y = pj * _M + xj * (1.0 - _M)
            ys.append(y)
            accs[j % 4] = accs[j % 4] + y * y
        ss = jnp.sum((accs[0] + accs[1]) + (accs[2] + accs[3]))
        sv = jnp.full((_L,), ss, jnp.float32)
        iv = plsc.bitcast(sv, jnp.int32)
        y0 = plsc.bitcast(jnp.int32(0x5F3759DF) - (iv >> 1), jnp.float32)
        y = y0
        for _ in range(4):
            y = y * (1.5 - 0.5 * sv * y * y)
        normv = sv * y
        scale = jnp.where(normv > 1e-12, y, 1e12)
        for j in range(_FEAT // _L):
            ptile[r, pl.ds(j * _L, _L)] = ys[j] * scale

        @pl.when(s + 2 < off)
        def _():
            fetch(s + 2, buf, sem_b)

    def pair(t, carry):
        s0 = 2 * t
        upd(s0, 0, semA)

        @pl.when(s0 + 1 < off)
        def _():
            upd(s0 + 1, 1, semB)

        return carry

    lax.fori_loop(0, (off + 1) // 2, pair, 0, unroll=False)

    @pl.when(wid < _NW - 1)
    def _():
        pltpu.sync_copy(ptile, out_hbm.at[pl.ds(base, _RPW)])

    @pl.when(wid == _NW - 1)
    def _():
        pltpu.sync_copy(ptile.at[pl.ds(0, _LAST_RPW)],
                        out_hbm.at[pl.ds(base, _LAST_RPW)])


def _loss_body(protos_ref, out_ref):
    protos = protos_ref[...]
    hi = protos.astype(jnp.bfloat16)
    lo = (protos - hi.astype(jnp.float32)).astype(jnp.bfloat16)
    dims = (((1,), (1,)), ((), ()))
    dot = lambda a, b: lax.dot_general(a, b, dims,
                                       preferred_element_type=jnp.float32)
    logits = (dot(hi, hi) + (dot(hi, lo) + dot(lo, hi))) * (1.0 / _TEMP)
    e = jnp.exp(logits)
    r = lax.broadcasted_iota(jnp.int32, (_N_CLS, _N_CLS), 0)
    c = lax.broadcasted_iota(jnp.int32, (_N_CLS, _N_CLS), 1)
    e = jnp.where(r == c, 0.0, e)
    s = jnp.sum(e, axis=1)
    mpn = jnp.log(s * (1.0 / (_N_CLS - 1)))
    out_ref[0, 0] = (_TEMP / _BASE_TEMP) * jnp.sum(mpn) * (1.0 / _N_CLS)


def kernel(features, prototypes, labels):
    ema = pl.kernel(
        _sc_ema_body,
        out_type=jax.ShapeDtypeStruct((_N_CLS, _FEAT), jnp.float32),
        mesh=plsc.VectorSubcoreMesh(core_axis_name="c", subcore_axis_name="s"),
        compiler_params=pltpu.CompilerParams(needs_layout_passes=False),
        scratch_types=[
            pltpu.VMEM((_BATCH + _L,), jnp.int32),
            pltpu.VMEM((_BATCH + _L,), jnp.int32),
            pltpu.VMEM((_BATCH + _L,), jnp.int32),
            pltpu.VMEM((2, _FEAT), jnp.float32),
            pltpu.VMEM((_RPW, _FEAT), jnp.float32),
            pltpu.VMEM_SHARED((_BATCH, _FEAT), jnp.float32),
            pltpu.SemaphoreType.DMA,
            pltpu.SemaphoreType.DMA,
            pltpu.SemaphoreType.DMA,
            pltpu.SemaphoreType.DMA,
        ],
    )
    protos_new = ema(features, prototypes, labels)

    out = pl.pallas_call(
        _loss_body,
        out_shape=jax.ShapeDtypeStruct((1, 1), jnp.float32),
        in_specs=[pl.BlockSpec(memory_space=pltpu.VMEM)],
        out_specs=pl.BlockSpec(memory_space=pltpu.SMEM),
    )(protos_new)
    return out[0, 0]
